# Initial kernel scaffold; baseline (speedup 1.0000x reference)
import jax
import jax.numpy as jnp
from jax.experimental import pallas as pl


def kernel(Xi, Xv, linW, linb, tables, W1, b1, W2, b2, g1, be1, rm1, rv1, g2, be2, rm2, rv2, bias):
    raise NotImplementedError("write your pallas kernel here")


# speedup vs baseline: 1.0439x; 1.0439x over previous
import functools

import jax
import jax.numpy as jnp
from jax import lax
from jax.experimental import pallas as pl
from jax.experimental.pallas import tpu as pltpu
from jax.experimental.pallas import tpu_sc as plsc

D = 16
NSPARSE = 10
NDENSE = 26


def _sc_gather(xis, tables):
    _, N = xis.shape
    NW = 32
    NB = N // NW
    C = 128
    NCH = NB // C

    mesh = plsc.VectorSubcoreMesh(core_axis_name="c", subcore_axis_name="s")

    @functools.partial(
        pl.kernel,
        mesh=mesh,
        compiler_params=pltpu.CompilerParams(use_tc_tiling_on_sc=False),
        out_type=jax.ShapeDtypeStruct((NSPARSE, N, D), jnp.float32),
        scratch_types=[
            pltpu.VMEM((NB,), jnp.int32),
            pltpu.VMEM((NB, D), jnp.float32),
            pltpu.SemaphoreType.DMA,
        ],
    )
    def k(xis_hbm, tab_hbm, out_hbm, idx_v, rows_v, sem):
        wid = lax.axis_index("s") * 2 + lax.axis_index("c")
        base = wid * NB
        for j in range(NSPARSE):
            pltpu.sync_copy(xis_hbm.at[j, pl.ds(base, NB)], idx_v)

            def chunk(c, _, j=j):
                s0 = c * C
                pltpu.async_copy(
                    tab_hbm.at[j].at[idx_v.at[pl.ds(s0, C)]],
                    rows_v.at[pl.ds(s0, C)],
                    sem,
                ).wait()
                return 0

            lax.fori_loop(0, NCH, chunk, 0)
            pltpu.sync_copy(rows_v, out_hbm.at[j, pl.ds(base, NB)])

    return k(xis, tables)


def _fold_body(*refs):
    with jax.default_matmul_precision("highest"):
        _fold_body_inner(*refs)


def _fold_body_inner(w1_ref, b1_ref, w2_ref, b2_ref, g1_ref, be1_ref, rm1_ref,
                     rv1_ref, g2_ref, be2_ref, rm2_ref, rv2_ref, wp_ref, cd_ref):
    a1 = g1_ref[:] * lax.rsqrt(rv1_ref[:] + 1e-5)
    a2 = g2_ref[:] * lax.rsqrt(rv2_ref[:] + 1e-5)
    u = a2 @ w2_ref[:]
    v = u * a1
    wp_ref[:] = v @ w1_ref[:] + 1.0
    cd = (b1_ref[:] * v + u * (be1_ref[:] - a1 * rm1_ref[:])
          + b2_ref[:] * a2 + (be2_ref[:] - rm2_ref[:] * a2))
    cd_ref[:] = jnp.sum(cd, axis=1, keepdims=True)


def _fold(W1, b1, W2, b2, g1, be1, rm1, rv1, g2, be2, rm2, rv2,
          interpret=False):
    r = lambda x: x.reshape(1, -1)
    wp, cd = pl.pallas_call(
        _fold_body,
        out_shape=[jax.ShapeDtypeStruct((1, 576), jnp.float32),
                   jax.ShapeDtypeStruct((1, 1), jnp.float32)],
        interpret=interpret,
    )(W1, r(b1), W2, r(b2), r(g1), r(be1), r(rm1), r(rv1),
      r(g2), r(be2), r(rm2), r(rv2))
    return wp.reshape(NDENSE + NSPARSE, D), cd


def _tc_body(*refs):
    with jax.default_matmul_precision("highest"):
        _tc_body_inner(*refs)


def _tc_body_inner(xi_ref, xv_ref, g_ref, lw_ref, lb_ref, wp_ref, cd_ref,
                   bias_ref, o_ref):
    wplus = wp_ref[:]
    c_deep = cd_ref[0, 0]
    xi = xi_ref[:].astype(jnp.float32)
    xv = xv_ref[:]
    xd = xi[:, :NDENSE]
    xvd = xv[:, :NDENSE]
    xq = xd * xvd

    lw = lw_ref[:]
    lb = lb_ref[:]
    wp_d = wplus[:NDENSE]
    s = xq @ lw + xvd @ lb
    aw = jnp.sum(lw * wp_d, axis=1)
    bw = jnp.sum(lb * wp_d, axis=1)
    p = xq @ aw + xvd @ bw
    wa = jnp.sum(lw * lw, axis=1)
    wb = jnp.sum(lw * lb, axis=1)
    wc = jnp.sum(lb * lb, axis=1)
    q = (xq * xq) @ wa + 2.0 * (xq * xvd) @ wb + (xvd * xvd) @ wc

    for j in range(NSPARSE):
        ej = g_ref[j] * xv[:, NDENSE + j][:, None]
        s = s + ej
        p = p + ej @ wplus[NDENSE + j]
        q = q + jnp.sum(ej * ej, axis=1)

    o_ref[:] = p + 0.5 * (jnp.sum(s * s, axis=1) - q) + c_deep + bias_ref[:]


def _tc_combine(Xi2, Xv, G, linW, linb, W1, b1, W2, b2, g1, be1, rm1, rv1,
                g2, be2, rm2, rv2, bias, interpret=False):
    N = Xi2.shape[0]
    Nb = 2048
    grid = N // Nb
    wplus, cdeep = _fold(W1, b1, W2, b2, g1, be1, rm1, rv1,
                         g2, be2, rm2, rv2, interpret=interpret)
    full = lambda shape: pl.BlockSpec(shape, lambda i: tuple(0 for _ in shape))
    return pl.pallas_call(
        _tc_body,
        grid=(grid,),
        in_specs=[
            pl.BlockSpec((Nb, NDENSE + NSPARSE), lambda i: (i, 0)),
            pl.BlockSpec((Nb, NDENSE + NSPARSE), lambda i: (i, 0)),
            pl.BlockSpec((NSPARSE, Nb, D), lambda i: (0, i, 0)),
            full(linW.shape), full(linb.shape),
            full(wplus.shape), full(cdeep.shape),
            pl.BlockSpec((Nb,), lambda i: (i,)),
        ],
        out_specs=pl.BlockSpec((Nb,), lambda i: (i,)),
        out_shape=jax.ShapeDtypeStruct((N,), jnp.float32),
        interpret=interpret,
    )(Xi2, Xv, G, linW, linb, wplus, cdeep, bias)


def kernel(Xi, Xv, linW, linb, tables, W1, b1, W2, b2, g1, be1, rm1, rv1,
           g2, be2, rm2, rv2, bias):
    N = Xi.shape[0]
    Xi2 = Xi.reshape(N, NDENSE + NSPARSE)
    xis = Xi2[:, NDENSE:].T.astype(jnp.int32)
    G = _sc_gather(xis, tables)
    return _tc_combine(Xi2, Xv, G, linW, linb, W1, b1, W2, b2,
                       g1, be1, rm1, rv1, g2, be2, rm2, rv2, bias)


# speedup vs baseline: 2.2298x; 2.1360x over previous
import functools

import jax
import jax.numpy as jnp
from jax import lax
from jax.experimental import pallas as pl
from jax.experimental.pallas import tpu as pltpu
from jax.experimental.pallas import tpu_sc as plsc

D = 16
NSPARSE = 10
NDENSE = 26
NF = NDENSE + NSPARSE


def _sc_gather(xiT, tabT):
    N = xiT.shape[1]
    NW = 32
    NB = N // NW
    C = 128
    NCH = NB // C
    NPLANES = NSPARSE * D
    TOT = NPLANES * NCH
    W = 16

    mesh = plsc.VectorSubcoreMesh(core_axis_name="c", subcore_axis_name="s")

    @functools.partial(
        pl.kernel,
        mesh=mesh,
        compiler_params=pltpu.CompilerParams(use_tc_tiling_on_sc=False,
                                             needs_layout_passes=False),
        out_type=jax.ShapeDtypeStruct((NSPARSE, D, N), jnp.float32),
        scratch_types=[
            pltpu.VMEM((NSPARSE, NB), jnp.int32),
            pltpu.VMEM((NSPARSE, D, NB), jnp.float32),
            pltpu.SemaphoreType.DMA,
        ],
    )
    def k(xi_hbm, tab_hbm, out_hbm, idx_v, rows_v, sem):
        wid = lax.axis_index("s") * 2 + lax.axis_index("c")
        base = wid * NB
        for j in range(NSPARSE):
            pltpu.sync_copy(xi_hbm.at[NDENSE + j, pl.ds(base, NB)],
                            idx_v.at[j])

        def xfer(t):
            jd = t // NCH
            c = t % NCH
            j = jd // D
            d = jd % D
            return (tab_hbm.at[j].at[d].at[idx_v.at[j].at[pl.ds(c * C, C)]],
                    rows_v.at[j].at[d].at[pl.ds(c * C, C)])

        def fire(t, _):
            src, dst = xfer(t)
            pltpu.async_copy(src, dst, sem)

            @pl.when(t >= W)
            def _():
                src0, dst0 = xfer(t - W)
                pltpu.make_async_copy(src0, dst0, sem).wait()
            return 0

        lax.fori_loop(0, TOT, fire, 0)

        def drain(t, _):
            src0, dst0 = xfer(t)
            pltpu.make_async_copy(src0, dst0, sem).wait()
            return 0

        lax.fori_loop(TOT - W, TOT, drain, 0)

        pltpu.sync_copy(rows_v, out_hbm.at[:, :, pl.ds(base, NB)])

    return k(xiT, tabT)


def _fold_body(*refs):
    with jax.default_matmul_precision("highest"):
        _fold_body_inner(*refs)


def _fold_body_inner(w1_ref, b1_ref, w2_ref, b2_ref, g1_ref, be1_ref, rm1_ref,
                     rv1_ref, g2_ref, be2_ref, rm2_ref, rv2_ref, wp_ref,
                     cd_ref):
    a1 = g1_ref[:] * lax.rsqrt(rv1_ref[:] + 1e-5)
    a2 = g2_ref[:] * lax.rsqrt(rv2_ref[:] + 1e-5)
    u = a2 @ w2_ref[:]
    v = u * a1
    wp_ref[:] = v @ w1_ref[:] + 1.0
    cd = (b1_ref[:] * v + u * (be1_ref[:] - a1 * rm1_ref[:])
          + b2_ref[:] * a2 + (be2_ref[:] - rm2_ref[:] * a2))
    cd_ref[:] = jnp.sum(cd, axis=1, keepdims=True)


def _fold(W1, b1, W2, b2, g1, be1, rm1, rv1, g2, be2, rm2, rv2,
          interpret=False):
    r = lambda x: x.reshape(1, -1)
    wp, cd = pl.pallas_call(
        _fold_body,
        out_shape=[jax.ShapeDtypeStruct((1, 576), jnp.float32),
                   jax.ShapeDtypeStruct((1, 1), jnp.float32)],
        interpret=interpret,
    )(W1, r(b1), W2, r(b2), r(g1), r(be1), r(rm1), r(rv1),
      r(g2), r(be2), r(rm2), r(rv2))
    return wp.reshape(NF, D), cd


def _tc_body(*refs):
    with jax.default_matmul_precision("highest"):
        _tc_body_inner(*refs)


def _tc_body_inner(xi_ref, xv_ref, g_ref, lw_ref, lb_ref, wp_ref, cd_ref,
                   bias_ref, o_ref):
    wplus = wp_ref[:]
    c_deep = cd_ref[0, 0]
    xiT = xi_ref[:].astype(jnp.float32)
    xvT = xv_ref[:]
    xdT = xiT[:NDENSE]
    xvdT = xvT[:NDENSE]
    xqT = xdT * xvdT

    lw = lw_ref[:]
    lb = lb_ref[:]
    wp_d = wplus[:NDENSE]
    dot00 = lambda a, b: lax.dot_general(a, b, (((0,), (0,)), ((), ())))
    sT = dot00(lw, xqT) + dot00(lb, xvdT)
    aw = jnp.sum(lw * wp_d, axis=1, keepdims=True)
    bw = jnp.sum(lb * wp_d, axis=1, keepdims=True)
    pT = dot00(aw, xqT) + dot00(bw, xvdT)
    wa = jnp.sum(lw * lw, axis=1, keepdims=True)
    wb = jnp.sum(lw * lb, axis=1, keepdims=True)
    wc = jnp.sum(lb * lb, axis=1, keepdims=True)
    qT = (dot00(wa, xqT * xqT) + 2.0 * dot00(wb, xqT * xvdT)
          + dot00(wc, xvdT * xvdT))

    for j in range(NSPARSE):
        ej = g_ref[j] * xvT[NDENSE + j][None, :]
        sT = sT + ej
        pT = pT + dot00(wplus[NDENSE + j][:, None], ej)
        qT = qT + jnp.sum(ej * ej, axis=0, keepdims=True)

    tot = pT + 0.5 * (jnp.sum(sT * sT, axis=0, keepdims=True) - qT) + c_deep
    o_ref[:] = tot[0] + bias_ref[:]


def _tc_combine(xiT, xvT, G, linW, linb, W1, b1, W2, b2, g1, be1, rm1, rv1,
                g2, be2, rm2, rv2, bias, interpret=False):
    N = xiT.shape[1]
    Nb = 2048
    grid = N // Nb
    wplus, cdeep = _fold(W1, b1, W2, b2, g1, be1, rm1, rv1,
                         g2, be2, rm2, rv2, interpret=interpret)
    full = lambda shape: pl.BlockSpec(shape, lambda i: tuple(0 for _ in shape))
    return pl.pallas_call(
        _tc_body,
        grid=(grid,),
        in_specs=[
            pl.BlockSpec((NF, Nb), lambda i: (0, i)),
            pl.BlockSpec((NF, Nb), lambda i: (0, i)),
            pl.BlockSpec((NSPARSE, D, Nb), lambda i: (0, 0, i)),
            full(linW.shape), full(linb.shape),
            full(wplus.shape), full(cdeep.shape),
            pl.BlockSpec((Nb,), lambda i: (i,)),
        ],
        out_specs=pl.BlockSpec((Nb,), lambda i: (i,)),
        out_shape=jax.ShapeDtypeStruct((N,), jnp.float32),
        interpret=interpret,
    )(xiT, xvT, G, linW, linb, wplus, cdeep, bias)


def kernel(Xi, Xv, linW, linb, tables, W1, b1, W2, b2, g1, be1, rm1, rv1,
           g2, be2, rm2, rv2, bias):
    N = Xi.shape[0]
    xiT = Xi.transpose(1, 2, 0).reshape(NF, N)
    xvT = Xv.T
    tabT = tables.transpose(0, 2, 1)
    G = _sc_gather(xiT.astype(jnp.int32), tabT)
    return _tc_combine(xiT, xvT, G, linW, linb, W1, b1, W2, b2,
                       g1, be1, rm1, rv1, g2, be2, rm2, rv2, bias)


# speedup vs baseline: 2.4933x; 1.1182x over previous
import functools

import jax
import jax.numpy as jnp
from jax import lax
from jax.experimental import pallas as pl
from jax.experimental.pallas import tpu as pltpu
from jax.experimental.pallas import tpu_sc as plsc

D = 16
NSPARSE = 10
NDENSE = 26
NF = NDENSE + NSPARSE


VP = 100352


def _detile_body(x_ref, o_ref):
    o_ref[0] = x_ref[0].reshape(D, -1, 128)


def _tc_detile(tabT):
    V = tabT.shape[2]
    BV = 14336
    return pl.pallas_call(
        _detile_body,
        grid=(NSPARSE, VP // BV),
        in_specs=[pl.BlockSpec((1, D, BV), lambda j, k: (j, 0, k))],
        out_specs=pl.BlockSpec((1, D, BV // 128, 128),
                               lambda j, k: (j, 0, k, 0)),
        out_shape=jax.ShapeDtypeStruct((NSPARSE, D, VP // 128, 128),
                                       jnp.float32),
    )(tabT)


def _sc_gather(xiT, tabT):
    N = xiT.shape[1]
    NW = 32
    NB = N // NW
    C = 128
    NCH = NB // C
    NPLANES = NSPARSE * D
    TOT = NPLANES * NCH
    W = 32

    mesh = plsc.VectorSubcoreMesh(core_axis_name="c", subcore_axis_name="s")

    @functools.partial(
        pl.kernel,
        mesh=mesh,
        compiler_params=pltpu.CompilerParams(use_tc_tiling_on_sc=False,
                                             needs_layout_passes=False),
        out_type=jax.ShapeDtypeStruct((NSPARSE, D, N), jnp.float32),
        scratch_types=[
            pltpu.VMEM((NSPARSE, NB), jnp.int32),
            pltpu.VMEM((NSPARSE, D, NB), jnp.float32),
            pltpu.SemaphoreType.DMA,
        ],
    )
    def k(xi_hbm, tab_hbm, out_hbm, idx_v, rows_v, sem):
        wid = lax.axis_index("s") * 2 + lax.axis_index("c")
        base = wid * NB
        for j in range(NSPARSE):
            pltpu.sync_copy(xi_hbm.at[NDENSE + j, pl.ds(base, NB)],
                            idx_v.at[j])

        def xfer(t):
            jd = t // NCH
            c = t % NCH
            j = jd // D
            d = jd % D
            return (tab_hbm.at[j].at[d].at[idx_v.at[j].at[pl.ds(c * C, C)]],
                    rows_v.at[j].at[d].at[pl.ds(c * C, C)])

        def fire(t, _):
            src, dst = xfer(t)
            pltpu.async_copy(src, dst, sem)

            @pl.when(t >= W)
            def _():
                src0, dst0 = xfer(t - W)
                pltpu.make_async_copy(src0, dst0, sem).wait()
            return 0

        lax.fori_loop(0, TOT, fire, 0)

        def drain(t, _):
            src0, dst0 = xfer(t)
            pltpu.make_async_copy(src0, dst0, sem).wait()
            return 0

        lax.fori_loop(TOT - W, TOT, drain, 0)

        pltpu.sync_copy(rows_v, out_hbm.at[:, :, pl.ds(base, NB)])

    return k(xiT, tabT)


def _fold_body(*refs):
    with jax.default_matmul_precision("highest"):
        _fold_body_inner(*refs)


def _fold_body_inner(w1_ref, b1_ref, w2_ref, b2_ref, g1_ref, be1_ref, rm1_ref,
                     rv1_ref, g2_ref, be2_ref, rm2_ref, rv2_ref, wp_ref,
                     cd_ref):
    a1 = g1_ref[:] * lax.rsqrt(rv1_ref[:] + 1e-5)
    a2 = g2_ref[:] * lax.rsqrt(rv2_ref[:] + 1e-5)
    u = a2 @ w2_ref[:]
    v = u * a1
    wp_ref[:] = v @ w1_ref[:] + 1.0
    cd = (b1_ref[:] * v + u * (be1_ref[:] - a1 * rm1_ref[:])
          + b2_ref[:] * a2 + (be2_ref[:] - rm2_ref[:] * a2))
    cd_ref[:] = jnp.sum(cd, axis=1, keepdims=True)


def _fold(W1, b1, W2, b2, g1, be1, rm1, rv1, g2, be2, rm2, rv2,
          interpret=False):
    r = lambda x: x.reshape(1, -1)
    wp, cd = pl.pallas_call(
        _fold_body,
        out_shape=[jax.ShapeDtypeStruct((1, 576), jnp.float32),
                   jax.ShapeDtypeStruct((1, 1), jnp.float32)],
        interpret=interpret,
    )(W1, r(b1), W2, r(b2), r(g1), r(be1), r(rm1), r(rv1),
      r(g2), r(be2), r(rm2), r(rv2))
    return wp.reshape(NF, D), cd


def _tc_body(*refs):
    with jax.default_matmul_precision("highest"):
        _tc_body_inner(*refs)


def _tc_body_inner(xi_ref, xv_ref, g_ref, lw_ref, lb_ref, wp_ref, cd_ref,
                   bias_ref, o_ref):
    wplus = wp_ref[:]
    c_deep = cd_ref[0, 0]
    xiT = xi_ref[:].astype(jnp.float32)
    xvT = xv_ref[:]
    xdT = xiT[:NDENSE]
    xvdT = xvT[:NDENSE]
    xqT = xdT * xvdT

    lw = lw_ref[:]
    lb = lb_ref[:]
    wp_d = wplus[:NDENSE]
    dot00 = lambda a, b: lax.dot_general(a, b, (((0,), (0,)), ((), ())))
    sT = dot00(lw, xqT) + dot00(lb, xvdT)
    aw = jnp.sum(lw * wp_d, axis=1, keepdims=True)
    bw = jnp.sum(lb * wp_d, axis=1, keepdims=True)
    pT = dot00(aw, xqT) + dot00(bw, xvdT)
    wa = jnp.sum(lw * lw, axis=1, keepdims=True)
    wb = jnp.sum(lw * lb, axis=1, keepdims=True)
    wc = jnp.sum(lb * lb, axis=1, keepdims=True)
    qT = (dot00(wa, xqT * xqT) + 2.0 * dot00(wb, xqT * xvdT)
          + dot00(wc, xvdT * xvdT))

    for j in range(NSPARSE):
        ej = g_ref[j] * xvT[NDENSE + j][None, :]
        sT = sT + ej
        pT = pT + dot00(wplus[NDENSE + j][:, None], ej)
        qT = qT + jnp.sum(ej * ej, axis=0, keepdims=True)

    tot = pT + 0.5 * (jnp.sum(sT * sT, axis=0, keepdims=True) - qT) + c_deep
    o_ref[:] = tot[0] + bias_ref[:]


def _tc_combine(xiT, xvT, G, linW, linb, W1, b1, W2, b2, g1, be1, rm1, rv1,
                g2, be2, rm2, rv2, bias, interpret=False):
    N = xiT.shape[1]
    Nb = 2048
    grid = N // Nb
    wplus, cdeep = _fold(W1, b1, W2, b2, g1, be1, rm1, rv1,
                         g2, be2, rm2, rv2, interpret=interpret)
    full = lambda shape: pl.BlockSpec(shape, lambda i: tuple(0 for _ in shape))
    return pl.pallas_call(
        _tc_body,
        grid=(grid,),
        in_specs=[
            pl.BlockSpec((NF, Nb), lambda i: (0, i)),
            pl.BlockSpec((NF, Nb), lambda i: (0, i)),
            pl.BlockSpec((NSPARSE, D, Nb), lambda i: (0, 0, i)),
            full(linW.shape), full(linb.shape),
            full(wplus.shape), full(cdeep.shape),
            pl.BlockSpec((Nb,), lambda i: (i,)),
        ],
        out_specs=pl.BlockSpec((Nb,), lambda i: (i,)),
        out_shape=jax.ShapeDtypeStruct((N,), jnp.float32),
        interpret=interpret,
    )(xiT, xvT, G, linW, linb, wplus, cdeep, bias)


def kernel(Xi, Xv, linW, linb, tables, W1, b1, W2, b2, g1, be1, rm1, rv1,
           g2, be2, rm2, rv2, bias):
    N = Xi.shape[0]
    xiT = Xi.transpose(1, 2, 0).reshape(NF, N)
    xvT = Xv.T
    tabT = tables.transpose(0, 2, 1)
    tabLin = _tc_detile(tabT).reshape(NSPARSE, D, VP)
    G = _sc_gather(xiT.astype(jnp.int32), tabLin)
    return _tc_combine(xiT, xvT, G, linW, linb, W1, b1, W2, b2,
                       g1, be1, rm1, rv1, g2, be2, rm2, rv2, bias)


# speedup vs baseline: 2.8699x; 1.1511x over previous
import functools

import jax
import jax.numpy as jnp
from jax import lax
from jax.experimental import pallas as pl
from jax.experimental.pallas import tpu as pltpu
from jax.experimental.pallas import tpu_sc as plsc

D = 16
NSPARSE = 10
NDENSE = 26
NF = NDENSE + NSPARSE


VP = 100352


def _detile_body(x_ref, o_ref):
    o_ref[0] = x_ref[0].reshape(D, -1, 128)


def _tc_detile(tabT):
    V = tabT.shape[2]
    BV = 50176
    return pl.pallas_call(
        _detile_body,
        grid=(NSPARSE, VP // BV),
        in_specs=[pl.BlockSpec((1, D, BV), lambda j, k: (j, 0, k))],
        out_specs=pl.BlockSpec((1, D, BV // 128, 128),
                               lambda j, k: (j, 0, k, 0)),
        out_shape=jax.ShapeDtypeStruct((NSPARSE, D, VP // 128, 128),
                                       jnp.float32),
    )(tabT)


def _sc_gather(xiT, tabT):
    N = xiT.shape[1]
    NW = 32
    NB = N // NW
    C = 128
    NCH = NB // C
    NPLANES = NSPARSE * D
    TOT = NPLANES * NCH
    W = 48

    mesh = plsc.VectorSubcoreMesh(core_axis_name="c", subcore_axis_name="s")

    @functools.partial(
        pl.kernel,
        mesh=mesh,
        compiler_params=pltpu.CompilerParams(use_tc_tiling_on_sc=False,
                                             needs_layout_passes=False),
        out_type=jax.ShapeDtypeStruct((NSPARSE, D, N), jnp.float32),
        scratch_types=[
            pltpu.VMEM((NSPARSE, NB), jnp.int32),
            pltpu.VMEM((NSPARSE, D, NB), jnp.float32),
            pltpu.SemaphoreType.DMA,
        ],
    )
    def k(xi_hbm, tab_hbm, out_hbm, idx_v, rows_v, sem):
        wid = lax.axis_index("s") * 2 + lax.axis_index("c")
        base = wid * NB
        for j in range(NSPARSE):
            pltpu.sync_copy(xi_hbm.at[NDENSE + j, pl.ds(base, NB)],
                            idx_v.at[j])

        def xfer(t):
            jd = t // NCH
            c = t % NCH
            j = jd // D
            d = jd % D
            return (tab_hbm.at[j].at[d].at[idx_v.at[j].at[pl.ds(c * C, C)]],
                    rows_v.at[j].at[d].at[pl.ds(c * C, C)])

        def fire(t, _):
            src, dst = xfer(t)
            pltpu.async_copy(src, dst, sem)

            @pl.when(t >= W)
            def _():
                src0, dst0 = xfer(t - W)
                pltpu.make_async_copy(src0, dst0, sem).wait()
            return 0

        lax.fori_loop(0, TOT, fire, 0)

        def drain(t, _):
            src0, dst0 = xfer(t)
            pltpu.make_async_copy(src0, dst0, sem).wait()
            return 0

        lax.fori_loop(TOT - W, TOT, drain, 0)

        pltpu.sync_copy(rows_v, out_hbm.at[:, :, pl.ds(base, NB)])

    return k(xiT, tabT)


def _fold_body(*refs):
    with jax.default_matmul_precision("highest"):
        _fold_body_inner(*refs)


def _fold_body_inner(w1_ref, b1_ref, w2_ref, b2_ref, g1_ref, be1_ref, rm1_ref,
                     rv1_ref, g2_ref, be2_ref, rm2_ref, rv2_ref, wp_ref,
                     cd_ref):
    a1 = g1_ref[:] * lax.rsqrt(rv1_ref[:] + 1e-5)
    a2 = g2_ref[:] * lax.rsqrt(rv2_ref[:] + 1e-5)
    u = a2 @ w2_ref[:]
    v = u * a1
    wp_ref[:] = v @ w1_ref[:] + 1.0
    cd = (b1_ref[:] * v + u * (be1_ref[:] - a1 * rm1_ref[:])
          + b2_ref[:] * a2 + (be2_ref[:] - rm2_ref[:] * a2))
    cd_ref[:] = jnp.sum(cd, axis=1, keepdims=True)


def _fold(W1, b1, W2, b2, g1, be1, rm1, rv1, g2, be2, rm2, rv2,
          interpret=False):
    r = lambda x: x.reshape(1, -1)
    wp, cd = pl.pallas_call(
        _fold_body,
        out_shape=[jax.ShapeDtypeStruct((1, 576), jnp.float32),
                   jax.ShapeDtypeStruct((1, 1), jnp.float32)],
        interpret=interpret,
    )(W1, r(b1), W2, r(b2), r(g1), r(be1), r(rm1), r(rv1),
      r(g2), r(be2), r(rm2), r(rv2))
    return wp.reshape(NF, D), cd


def _tc_body(*refs):
    with jax.default_matmul_precision("highest"):
        _tc_body_inner(*refs)


def _tc_body_inner(xi_ref, xv_ref, g_ref, lw_ref, lb_ref, wp_ref, cd_ref,
                   bias_ref, o_ref):
    wplus = wp_ref[:]
    c_deep = cd_ref[0, 0]
    xiT = xi_ref[:].astype(jnp.float32)
    xvT = xv_ref[:]
    xdT = xiT[:NDENSE]
    xvdT = xvT[:NDENSE]
    xqT = xdT * xvdT

    lw = lw_ref[:]
    lb = lb_ref[:]
    wp_d = wplus[:NDENSE]
    dot00 = lambda a, b: lax.dot_general(a, b, (((0,), (0,)), ((), ())))
    sT = dot00(lw, xqT) + dot00(lb, xvdT)
    aw = jnp.sum(lw * wp_d, axis=1, keepdims=True)
    bw = jnp.sum(lb * wp_d, axis=1, keepdims=True)
    pT = dot00(aw, xqT) + dot00(bw, xvdT)
    wa = jnp.sum(lw * lw, axis=1, keepdims=True)
    wb = jnp.sum(lw * lb, axis=1, keepdims=True)
    wc = jnp.sum(lb * lb, axis=1, keepdims=True)
    qT = (dot00(wa, xqT * xqT) + 2.0 * dot00(wb, xqT * xvdT)
          + dot00(wc, xvdT * xvdT))

    for j in range(NSPARSE):
        ej = g_ref[j] * xvT[NDENSE + j][None, :]
        sT = sT + ej
        pT = pT + dot00(wplus[NDENSE + j][:, None], ej)
        qT = qT + jnp.sum(ej * ej, axis=0, keepdims=True)

    tot = pT + 0.5 * (jnp.sum(sT * sT, axis=0, keepdims=True) - qT) + c_deep
    o_ref[:] = tot[0] + bias_ref[:]


def _tc_combine(xiT, xvT, G, linW, linb, W1, b1, W2, b2, g1, be1, rm1, rv1,
                g2, be2, rm2, rv2, bias, interpret=False):
    N = xiT.shape[1]
    Nb = 2048
    grid = N // Nb
    wplus, cdeep = _fold(W1, b1, W2, b2, g1, be1, rm1, rv1,
                         g2, be2, rm2, rv2, interpret=interpret)
    full = lambda shape: pl.BlockSpec(shape, lambda i: tuple(0 for _ in shape))
    return pl.pallas_call(
        _tc_body,
        grid=(grid,),
        in_specs=[
            pl.BlockSpec((NF, Nb), lambda i: (0, i)),
            pl.BlockSpec((NF, Nb), lambda i: (0, i)),
            pl.BlockSpec((NSPARSE, D, Nb), lambda i: (0, 0, i)),
            full(linW.shape), full(linb.shape),
            full(wplus.shape), full(cdeep.shape),
            pl.BlockSpec((Nb,), lambda i: (i,)),
        ],
        out_specs=pl.BlockSpec((Nb,), lambda i: (i,)),
        out_shape=jax.ShapeDtypeStruct((N,), jnp.float32),
        interpret=interpret,
    )(xiT, xvT, G, linW, linb, wplus, cdeep, bias)


def kernel(Xi, Xv, linW, linb, tables, W1, b1, W2, b2, g1, be1, rm1, rv1,
           g2, be2, rm2, rv2, bias):
    N = Xi.shape[0]
    xiT = Xi.transpose(1, 2, 0).reshape(NF, N)
    xvT = Xv.T
    tabT = tables.transpose(0, 2, 1)
    tabLin = _tc_detile(tabT).reshape(NSPARSE, D, VP)
    G = _sc_gather(xiT.astype(jnp.int32), tabLin)
    return _tc_combine(xiT, xvT, G, linW, linb, W1, b1, W2, b2,
                       g1, be1, rm1, rv1, g2, be2, rm2, rv2, bias)
